# baseline (speedup 1.0000x reference)
import jax
import jax.numpy as jnp
from jax import lax
from jax.experimental import pallas as pl
from jax.experimental.pallas import tpu as pltpu
from jax.experimental.pallas import tpu_sc as plsc

B = 4096
S = 200
D = 32
L = 16
NC = 2
NS = 16
NW = NC * NS
RPW = B // NW
C0 = 128
C1 = S - C0
INV_S = 1.0 / S
INV_D = 1.0 / (S - 1)


VROWS = 1000000
CH_TOK = 512
N_CH = 1954
LAST_TOK = VROWS - (N_CH - 1) * CH_TOK
KMAX = (N_CH + NW - 1) // NW


def _relayout_body(embT_hbm, out_hbm, inb0, inb1, row0, row1,
                   sem_i0, sem_i1, sem_o0, sem_o1):
    cid = lax.axis_index("c")
    sid = lax.axis_index("s")
    wid = sid * NC + cid

    iota = lax.iota(jnp.int32, L)
    jb_lo = lax.shift_right_logical(iota, 3)
    jb_hi = jb_lo + 2
    j_lane = jnp.bitwise_and(iota, 7)

    TPC = CH_TOK // 128

    def stage(c, inb, sem):
        @pl.when(c < N_CH - 1)
        def _():
            base = pl.multiple_of(c * CH_TOK, CH_TOK)
            for jb in range(4):
                for t in range(TPC):
                    pltpu.async_copy(
                        embT_hbm.at[pl.ds(jb * 8, 8), pl.ds(base + t * 128, 128)],
                        inb.at[jb, t], sem)

        @pl.when(c == N_CH - 1)
        def _():
            base = pl.multiple_of(c * CH_TOK, CH_TOK)
            for jb in range(4):
                pltpu.async_copy(
                    embT_hbm.at[pl.ds(jb * 8, 8), pl.ds(base, 128)],
                    inb.at[jb, 0], sem)

    def drain(c, inb, sem):
        @pl.when(c < N_CH - 1)
        def _():
            for jb in range(4):
                for t in range(TPC):
                    pltpu.make_async_copy(
                        embT_hbm.at[pl.ds(0, 8), pl.ds(0, 128)],
                        inb.at[jb, t], sem).wait()

        @pl.when(c == N_CH - 1)
        def _():
            for jb in range(4):
                pltpu.make_async_copy(
                    embT_hbm.at[pl.ds(0, 8), pl.ds(0, 128)],
                    inb.at[jb, 0], sem).wait()

    def transpose_chunk(c, inb, row, sem_o):
        for t in range(TPC):
            tvec = jnp.full((L,), t, jnp.int32)

            def tok_body(i, carry, t=t, tvec=tvec):
                di = jnp.full((L,), i, jnp.int32)
                g1 = plsc.load_gather(inb, [jb_lo, tvec, j_lane, di])
                g2 = plsc.load_gather(inb, [jb_hi, tvec, j_lane, di])
                row[pl.ds((t * 128 + i) * D, L)] = g1
                row[pl.ds((t * 128 + i) * D + L, L)] = g2
                return carry

            lax.fori_loop(0, 128, tok_body, 0, unroll=4)

        @pl.when(c < N_CH - 1)
        def _():
            pltpu.async_copy(
                row, out_hbm.at[pl.ds(c * (CH_TOK * D), CH_TOK * D)], sem_o)

        @pl.when(c == N_CH - 1)
        def _():
            pltpu.async_copy(
                row.at[pl.ds(0, LAST_TOK * D)],
                out_hbm.at[pl.ds(c * (CH_TOK * D), LAST_TOK * D)], sem_o)

    def drain_out(c, row, sem_o):
        @pl.when(c < N_CH - 1)
        def _():
            pltpu.make_async_copy(
                row, out_hbm.at[pl.ds(0, CH_TOK * D)], sem_o).wait()

        @pl.when(c == N_CH - 1)
        def _():
            pltpu.make_async_copy(
                row.at[pl.ds(0, LAST_TOK * D)],
                out_hbm.at[pl.ds(0, LAST_TOK * D)], sem_o).wait()

    stage(wid, inb0, sem_i0)
    npairs = (KMAX + 1) // 2

    def k_body(k, carry):
        c0 = wid + NW * (2 * k)
        c1 = wid + NW * (2 * k + 1)
        c2 = wid + NW * (2 * k + 2)
        cp0 = wid + NW * (2 * k - 2)
        cp1 = wid + NW * (2 * k - 1)

        @pl.when(c1 < N_CH)
        def _():
            stage(c1, inb1, sem_i1)

        @pl.when((k > 0) & (cp0 < N_CH))
        def _():
            drain_out(cp0, row0, sem_o0)

        @pl.when(c0 < N_CH)
        def _():
            drain(c0, inb0, sem_i0)
            transpose_chunk(c0, inb0, row0, sem_o0)

        @pl.when(c2 < N_CH)
        def _():
            stage(c2, inb0, sem_i0)

        @pl.when((k > 0) & (cp1 < N_CH))
        def _():
            drain_out(cp1, row1, sem_o1)

        @pl.when(c1 < N_CH)
        def _():
            drain(c1, inb1, sem_i1)
            transpose_chunk(c1, inb1, row1, sem_o1)

        return carry

    lax.fori_loop(0, npairs, k_body, 0)

    @pl.when(wid + NW * (KMAX - 2) < N_CH)
    def _():
        drain_out(wid + NW * (KMAX - 2), row0, sem_o0)

    @pl.when(wid + NW * (KMAX - 1) < N_CH)
    def _():
        drain_out(wid + NW * (KMAX - 1), row1, sem_o1)


def _relayout(embT):
    mesh = plsc.VectorSubcoreMesh(
        core_axis_name="c", subcore_axis_name="s", num_cores=NC, num_subcores=NS)
    run = pl.kernel(
        _relayout_body,
        out_type=jax.ShapeDtypeStruct((VROWS * D,), jnp.float32),
        mesh=mesh,
        compiler_params=pltpu.CompilerParams(
            needs_layout_passes=False, use_tc_tiling_on_sc=True),
        scratch_types=[
            pltpu.VMEM((4, CH_TOK // 128, 8, 128), jnp.float32),
            pltpu.VMEM((4, CH_TOK // 128, 8, 128), jnp.float32),
            pltpu.VMEM((CH_TOK * D,), jnp.float32),
            pltpu.VMEM((CH_TOK * D,), jnp.float32),
            pltpu.SemaphoreType.DMA,
            pltpu.SemaphoreType.DMA,
            pltpu.SemaphoreType.DMA,
            pltpu.SemaphoreType.DMA,
        ],
    )
    return run(embT)


def _sc_body(tf_hbm, tg_hbm, emb_hbm, z_hbm, out_hbm,
             tf_v, tg_v, z_v, buf_f0, buf_g0, buf_f1, buf_g1, out_v,
             sem0, sem1):
    cid = lax.axis_index("c")
    sid = lax.axis_index("s")
    wid = sid * NC + cid
    base = wid * RPW

    pltpu.sync_copy(tf_hbm.at[pl.ds(base, RPW), :], tf_v)
    pltpu.sync_copy(tg_hbm.at[pl.ds(base, RPW), :], tg_v)
    pltpu.sync_copy(z_hbm, z_v)
    z0 = z_v[pl.ds(0, L)]
    z1 = z_v[pl.ds(L, L)]

    iota = lax.iota(jnp.int32, L)
    lane0 = iota == 0
    bfly = [jnp.bitwise_xor(iota, k) for k in (8, 4, 2, 1)]

    def issue_pair(r, buf_f, buf_g, sem):
        pltpu.async_copy(
            emb_hbm.at[tf_v.at[r, pl.ds(0, C0)]], buf_f.at[pl.ds(0, C0), :], sem)
        pltpu.async_copy(
            emb_hbm.at[tf_v.at[r, pl.ds(C0, C1)]], buf_f.at[pl.ds(C0, C1), :], sem)
        pltpu.async_copy(
            emb_hbm.at[tg_v.at[r, pl.ds(0, C0)]], buf_g.at[pl.ds(0, C0), :], sem)
        pltpu.async_copy(
            emb_hbm.at[tg_v.at[r, pl.ds(C0, C1)]], buf_g.at[pl.ds(C0, C1), :], sem)

    def drain_pair(buf_f, buf_g, sem):
        pltpu.make_async_copy(emb_hbm.at[pl.ds(0, S), :], buf_f, sem).wait()
        pltpu.make_async_copy(emb_hbm.at[pl.ds(0, S), :], buf_g, sem).wait()

    def compute_row(r, buf_f, buf_g):
        f0a = buf_f[0, pl.ds(0, L)]
        f0b = buf_f[0, pl.ds(L, L)]
        g0a = buf_g[0, pl.ds(0, L)]
        g0b = buf_g[0, pl.ds(L, L)]
        zv = jnp.zeros((L,), jnp.float32)

        def step(s, carry):
            (fsa, fsb, fqa, fqb, fca, fcb, fpa, fpb,
             gsa, gsb, gqa, gqb, gca, gcb, gpa, gpb) = carry
            fva = buf_f[s, pl.ds(0, L)]
            fvb = buf_f[s, pl.ds(L, L)]
            gva = buf_g[s, pl.ds(0, L)]
            gvb = buf_g[s, pl.ds(L, L)]
            return (fsa + fva, fsb + fvb, fqa + fva * fva, fqb + fvb * fvb,
                    fca + fva * fpa, fcb + fvb * fpb, fva, fvb,
                    gsa + gva, gsb + gvb, gqa + gva * gva, gqb + gvb * gvb,
                    gca + gva * gpa, gcb + gvb * gpb, gva, gvb)

        (fsa, fsb, fqa, fqb, fca, fcb, fpa, fpb,
         gsa, gsb, gqa, gqb, gca, gcb, gpa, gpb) = lax.fori_loop(
            0, S, step,
            (zv, zv, zv, zv, zv, zv, f0a, f0b,
             zv, zv, zv, zv, zv, zv, g0a, g0b), unroll=8)

        inv_s = jnp.float32(INV_S)
        inv_d = jnp.float32(INV_D)
        acc = jnp.zeros((L,), jnp.float32)
        for (fs, fq, fc, fp, f0, gs, gq, gc, gp, g0, z) in (
                (fsa, fqa, fca, fpa, f0a, gsa, gqa, gca, gpa, g0a, z0),
                (fsb, fqb, fcb, fpb, f0b, gsb, gqb, gcb, gpb, g0b, z1)):
            sd = (fs - gs) * inv_s
            ed = (fq - gq) * inv_s - 2.0 * z * sd
            dmd = ((fp - f0) - (gp - g0)) * inv_d
            fdq = 2.0 * fq + f0 * f0 - fp * fp - 2.0 * fc
            gdq = 2.0 * gq + g0 * g0 - gp * gp - 2.0 * gc
            ded = (fdq - gdq) * inv_d
            acc = acc + sd * sd + ed * ed + dmd * dmd + ded * ded
        for idx in bfly:
            acc = acc + acc.at[idx].get(mode="promise_in_bounds")
        dist = acc * jnp.float32(1.0 / (4 * D))
        plsc.store_scatter(out_v, [jnp.full((L,), r, jnp.int32)], dist, mask=lane0)

    issue_pair(0, buf_f0, buf_g0, sem0)

    def pair_body(rr, carry):
        r0 = 2 * rr
        issue_pair(r0 + 1, buf_f1, buf_g1, sem1)
        drain_pair(buf_f0, buf_g0, sem0)
        compute_row(r0, buf_f0, buf_g0)

        @pl.when(rr < RPW // 2 - 1)
        def _():
            issue_pair(r0 + 2, buf_f0, buf_g0, sem0)

        drain_pair(buf_f1, buf_g1, sem1)
        compute_row(r0 + 1, buf_f1, buf_g1)
        return carry

    lax.fori_loop(0, RPW // 2, pair_body, 0)
    pltpu.sync_copy(out_v, out_hbm.at[pl.ds(base, RPW)])


def kernel(tokens_f, tokens_g, embedding, state_zero):
    emb_lin = _relayout(embedding.T).reshape(VROWS, D)
    mesh = plsc.VectorSubcoreMesh(
        core_axis_name="c", subcore_axis_name="s", num_cores=NC, num_subcores=NS)
    run = pl.kernel(
        _sc_body,
        out_type=jax.ShapeDtypeStruct((B,), jnp.float32),
        mesh=mesh,
        compiler_params=pltpu.CompilerParams(
            needs_layout_passes=False, use_tc_tiling_on_sc=False),
        scratch_types=[
            pltpu.VMEM((RPW, S), jnp.int32),
            pltpu.VMEM((RPW, S), jnp.int32),
            pltpu.VMEM((D,), jnp.float32),
            pltpu.VMEM((S, D), jnp.float32),
            pltpu.VMEM((S, D), jnp.float32),
            pltpu.VMEM((S, D), jnp.float32),
            pltpu.VMEM((S, D), jnp.float32),
            pltpu.VMEM((RPW,), jnp.float32),
            pltpu.SemaphoreType.DMA,
            pltpu.SemaphoreType.DMA,
        ],
    )
    return run(tokens_f.astype(jnp.int32), tokens_g.astype(jnp.int32),
               emb_lin, state_zero)


# speedup vs baseline: 1.5981x; 1.5981x over previous
import jax
import jax.numpy as jnp
from jax import lax
from jax.experimental import pallas as pl
from jax.experimental.pallas import tpu as pltpu
from jax.experimental.pallas import tpu_sc as plsc

B = 4096
S = 200
D = 32
L = 16
NC = 2
NS = 16
NW = NC * NS
RPW = B // NW
C0 = 128
C1 = S - C0
INV_S = 1.0 / S
INV_D = 1.0 / (S - 1)


VROWS = 1000000
CH_TOK = 512
N_CH = 1954
LAST_TOK = VROWS - (N_CH - 1) * CH_TOK
KMAX = (N_CH + NW - 1) // NW


def _relayout_body(embT_hbm, out_hbm, inb0, inb1, row0, row1,
                   sem_i0, sem_i1, sem_o0, sem_o1):
    cid = lax.axis_index("c")
    sid = lax.axis_index("s")
    wid = sid * NC + cid

    iota = lax.iota(jnp.int32, L)
    jb_lo = lax.shift_right_logical(iota, 3)
    jb_hi = jb_lo + 2
    j_lane = jnp.bitwise_and(iota, 7)

    TPC = CH_TOK // 128

    def stage(c, inb, sem):
        @pl.when(c < N_CH - 1)
        def _():
            base = pl.multiple_of(c * CH_TOK, CH_TOK)
            for jb in range(4):
                for t in range(TPC):
                    pltpu.async_copy(
                        embT_hbm.at[pl.ds(jb * 8, 8), pl.ds(base + t * 128, 128)],
                        inb.at[jb, t], sem)

        @pl.when(c == N_CH - 1)
        def _():
            base = pl.multiple_of(c * CH_TOK, CH_TOK)
            for jb in range(4):
                pltpu.async_copy(
                    embT_hbm.at[pl.ds(jb * 8, 8), pl.ds(base, 128)],
                    inb.at[jb, 0], sem)

    def drain(c, inb, sem):
        @pl.when(c < N_CH - 1)
        def _():
            for jb in range(4):
                for t in range(TPC):
                    pltpu.make_async_copy(
                        embT_hbm.at[pl.ds(0, 8), pl.ds(0, 128)],
                        inb.at[jb, t], sem).wait()

        @pl.when(c == N_CH - 1)
        def _():
            for jb in range(4):
                pltpu.make_async_copy(
                    embT_hbm.at[pl.ds(0, 8), pl.ds(0, 128)],
                    inb.at[jb, 0], sem).wait()

    RS = D + 1
    iota_rs = iota * RS

    def transpose_chunk(c, inb, row, sem_o):
        for t in range(TPC):
            def g_body(g, carry, t=t):
                gbase = (t * 128 + g * L) * RS
                for jb in range(4):
                    for j in range(8):
                        v = inb[jb, t, j, pl.ds(g * L, L)]
                        idx = iota_rs + (gbase + jb * 8 + j)
                        plsc.store_scatter(row, [idx], v)
                return carry

            lax.fori_loop(0, 128 // L, g_body, 0)

        def c_body(i, carry):
            v1 = row[pl.ds(i * RS, L)]
            v2 = row[pl.ds(i * RS + L, L)]
            row[pl.ds(i * D, L)] = v1
            row[pl.ds(i * D + L, L)] = v2
            return carry

        lax.fori_loop(0, CH_TOK, c_body, 0, unroll=4)

        @pl.when(c < N_CH - 1)
        def _():
            pltpu.async_copy(
                row.at[pl.ds(0, CH_TOK * D)],
                out_hbm.at[pl.ds(c * (CH_TOK * D), CH_TOK * D)], sem_o)

        @pl.when(c == N_CH - 1)
        def _():
            pltpu.async_copy(
                row.at[pl.ds(0, LAST_TOK * D)],
                out_hbm.at[pl.ds(c * (CH_TOK * D), LAST_TOK * D)], sem_o)

    def drain_out(c, row, sem_o):
        @pl.when(c < N_CH - 1)
        def _():
            pltpu.make_async_copy(
                row.at[pl.ds(0, CH_TOK * D)],
                out_hbm.at[pl.ds(0, CH_TOK * D)], sem_o).wait()

        @pl.when(c == N_CH - 1)
        def _():
            pltpu.make_async_copy(
                row.at[pl.ds(0, LAST_TOK * D)],
                out_hbm.at[pl.ds(0, LAST_TOK * D)], sem_o).wait()

    stage(wid, inb0, sem_i0)
    npairs = (KMAX + 1) // 2

    def k_body(k, carry):
        c0 = wid + NW * (2 * k)
        c1 = wid + NW * (2 * k + 1)
        c2 = wid + NW * (2 * k + 2)
        cp0 = wid + NW * (2 * k - 2)
        cp1 = wid + NW * (2 * k - 1)

        @pl.when(c1 < N_CH)
        def _():
            stage(c1, inb1, sem_i1)

        @pl.when((k > 0) & (cp0 < N_CH))
        def _():
            drain_out(cp0, row0, sem_o0)

        @pl.when(c0 < N_CH)
        def _():
            drain(c0, inb0, sem_i0)
            transpose_chunk(c0, inb0, row0, sem_o0)

        @pl.when(c2 < N_CH)
        def _():
            stage(c2, inb0, sem_i0)

        @pl.when((k > 0) & (cp1 < N_CH))
        def _():
            drain_out(cp1, row1, sem_o1)

        @pl.when(c1 < N_CH)
        def _():
            drain(c1, inb1, sem_i1)
            transpose_chunk(c1, inb1, row1, sem_o1)

        return carry

    lax.fori_loop(0, npairs, k_body, 0)

    @pl.when(wid + NW * (KMAX - 2) < N_CH)
    def _():
        drain_out(wid + NW * (KMAX - 2), row0, sem_o0)

    @pl.when(wid + NW * (KMAX - 1) < N_CH)
    def _():
        drain_out(wid + NW * (KMAX - 1), row1, sem_o1)


def _relayout(embT):
    mesh = plsc.VectorSubcoreMesh(
        core_axis_name="c", subcore_axis_name="s", num_cores=NC, num_subcores=NS)
    run = pl.kernel(
        _relayout_body,
        out_type=jax.ShapeDtypeStruct((VROWS * D,), jnp.float32),
        mesh=mesh,
        compiler_params=pltpu.CompilerParams(
            needs_layout_passes=False, use_tc_tiling_on_sc=True),
        scratch_types=[
            pltpu.VMEM((4, CH_TOK // 128, 8, 128), jnp.float32),
            pltpu.VMEM((4, CH_TOK // 128, 8, 128), jnp.float32),
            pltpu.VMEM((CH_TOK * (D + 1),), jnp.float32),
            pltpu.VMEM((CH_TOK * (D + 1),), jnp.float32),
            pltpu.SemaphoreType.DMA,
            pltpu.SemaphoreType.DMA,
            pltpu.SemaphoreType.DMA,
            pltpu.SemaphoreType.DMA,
        ],
    )
    return run(embT)


def _sc_body(tf_hbm, tg_hbm, emb_hbm, z_hbm, out_hbm,
             tf_v, tg_v, z_v, buf_f0, buf_g0, buf_f1, buf_g1, out_v,
             sem0, sem1):
    cid = lax.axis_index("c")
    sid = lax.axis_index("s")
    wid = sid * NC + cid
    base = wid * RPW

    pltpu.sync_copy(tf_hbm.at[pl.ds(base, RPW), :], tf_v)
    pltpu.sync_copy(tg_hbm.at[pl.ds(base, RPW), :], tg_v)
    pltpu.sync_copy(z_hbm, z_v)
    z0 = z_v[pl.ds(0, L)]
    z1 = z_v[pl.ds(L, L)]

    iota = lax.iota(jnp.int32, L)
    lane0 = iota == 0
    bfly = [jnp.bitwise_xor(iota, k) for k in (8, 4, 2, 1)]

    def issue_pair(r, buf_f, buf_g, sem):
        pltpu.async_copy(
            emb_hbm.at[tf_v.at[r, pl.ds(0, C0)]], buf_f.at[pl.ds(0, C0), :], sem)
        pltpu.async_copy(
            emb_hbm.at[tf_v.at[r, pl.ds(C0, C1)]], buf_f.at[pl.ds(C0, C1), :], sem)
        pltpu.async_copy(
            emb_hbm.at[tg_v.at[r, pl.ds(0, C0)]], buf_g.at[pl.ds(0, C0), :], sem)
        pltpu.async_copy(
            emb_hbm.at[tg_v.at[r, pl.ds(C0, C1)]], buf_g.at[pl.ds(C0, C1), :], sem)

    def drain_pair(buf_f, buf_g, sem):
        pltpu.make_async_copy(emb_hbm.at[pl.ds(0, S), :], buf_f, sem).wait()
        pltpu.make_async_copy(emb_hbm.at[pl.ds(0, S), :], buf_g, sem).wait()

    def compute_row(r, buf_f, buf_g):
        f0a = buf_f[0, pl.ds(0, L)]
        f0b = buf_f[0, pl.ds(L, L)]
        g0a = buf_g[0, pl.ds(0, L)]
        g0b = buf_g[0, pl.ds(L, L)]
        zv = jnp.zeros((L,), jnp.float32)

        def step(s, carry):
            (fsa, fsb, fqa, fqb, fca, fcb, fpa, fpb,
             gsa, gsb, gqa, gqb, gca, gcb, gpa, gpb) = carry
            fva = buf_f[s, pl.ds(0, L)]
            fvb = buf_f[s, pl.ds(L, L)]
            gva = buf_g[s, pl.ds(0, L)]
            gvb = buf_g[s, pl.ds(L, L)]
            return (fsa + fva, fsb + fvb, fqa + fva * fva, fqb + fvb * fvb,
                    fca + fva * fpa, fcb + fvb * fpb, fva, fvb,
                    gsa + gva, gsb + gvb, gqa + gva * gva, gqb + gvb * gvb,
                    gca + gva * gpa, gcb + gvb * gpb, gva, gvb)

        (fsa, fsb, fqa, fqb, fca, fcb, fpa, fpb,
         gsa, gsb, gqa, gqb, gca, gcb, gpa, gpb) = lax.fori_loop(
            0, S, step,
            (zv, zv, zv, zv, zv, zv, f0a, f0b,
             zv, zv, zv, zv, zv, zv, g0a, g0b), unroll=8)

        inv_s = jnp.float32(INV_S)
        inv_d = jnp.float32(INV_D)
        acc = jnp.zeros((L,), jnp.float32)
        for (fs, fq, fc, fp, f0, gs, gq, gc, gp, g0, z) in (
                (fsa, fqa, fca, fpa, f0a, gsa, gqa, gca, gpa, g0a, z0),
                (fsb, fqb, fcb, fpb, f0b, gsb, gqb, gcb, gpb, g0b, z1)):
            sd = (fs - gs) * inv_s
            ed = (fq - gq) * inv_s - 2.0 * z * sd
            dmd = ((fp - f0) - (gp - g0)) * inv_d
            fdq = 2.0 * fq + f0 * f0 - fp * fp - 2.0 * fc
            gdq = 2.0 * gq + g0 * g0 - gp * gp - 2.0 * gc
            ded = (fdq - gdq) * inv_d
            acc = acc + sd * sd + ed * ed + dmd * dmd + ded * ded
        for idx in bfly:
            acc = acc + acc.at[idx].get(mode="promise_in_bounds")
        dist = acc * jnp.float32(1.0 / (4 * D))
        plsc.store_scatter(out_v, [jnp.full((L,), r, jnp.int32)], dist, mask=lane0)

    issue_pair(0, buf_f0, buf_g0, sem0)

    def pair_body(rr, carry):
        r0 = 2 * rr
        issue_pair(r0 + 1, buf_f1, buf_g1, sem1)
        drain_pair(buf_f0, buf_g0, sem0)
        compute_row(r0, buf_f0, buf_g0)

        @pl.when(rr < RPW // 2 - 1)
        def _():
            issue_pair(r0 + 2, buf_f0, buf_g0, sem0)

        drain_pair(buf_f1, buf_g1, sem1)
        compute_row(r0 + 1, buf_f1, buf_g1)
        return carry

    lax.fori_loop(0, RPW // 2, pair_body, 0)
    pltpu.sync_copy(out_v, out_hbm.at[pl.ds(base, RPW)])


def kernel(tokens_f, tokens_g, embedding, state_zero):
    emb_lin = _relayout(embedding.T).reshape(VROWS, D)
    mesh = plsc.VectorSubcoreMesh(
        core_axis_name="c", subcore_axis_name="s", num_cores=NC, num_subcores=NS)
    run = pl.kernel(
        _sc_body,
        out_type=jax.ShapeDtypeStruct((B,), jnp.float32),
        mesh=mesh,
        compiler_params=pltpu.CompilerParams(
            needs_layout_passes=False, use_tc_tiling_on_sc=False),
        scratch_types=[
            pltpu.VMEM((RPW, S), jnp.int32),
            pltpu.VMEM((RPW, S), jnp.int32),
            pltpu.VMEM((D,), jnp.float32),
            pltpu.VMEM((S, D), jnp.float32),
            pltpu.VMEM((S, D), jnp.float32),
            pltpu.VMEM((S, D), jnp.float32),
            pltpu.VMEM((S, D), jnp.float32),
            pltpu.VMEM((RPW,), jnp.float32),
            pltpu.SemaphoreType.DMA,
            pltpu.SemaphoreType.DMA,
        ],
    )
    return run(tokens_f.astype(jnp.int32), tokens_g.astype(jnp.int32),
               emb_lin, state_zero)


# speedup vs baseline: 1.6141x; 1.0100x over previous
import jax
import jax.numpy as jnp
from jax import lax
from jax.experimental import pallas as pl
from jax.experimental.pallas import tpu as pltpu
from jax.experimental.pallas import tpu_sc as plsc

B = 4096
S = 200
D = 32
L = 16
NC = 2
NS = 16
NW = NC * NS
RPW = B // NW
C0 = 128
C1 = S - C0
INV_S = 1.0 / S
INV_D = 1.0 / (S - 1)


VROWS = 1000000
CH_TOK = 512
N_CH = 1954
LAST_TOK = VROWS - (N_CH - 1) * CH_TOK
KMAX = (N_CH + NW - 1) // NW


def _relayout_body(embT_hbm, out_hbm, inb0, inb1, row0, row1,
                   sem_i0, sem_i1, sem_o0, sem_o1):
    cid = lax.axis_index("c")
    sid = lax.axis_index("s")
    wid = sid * NC + cid

    iota = lax.iota(jnp.int32, L)
    jb_lo = lax.shift_right_logical(iota, 3)
    jb_hi = jb_lo + 2
    j_lane = jnp.bitwise_and(iota, 7)

    def stage(c, inb, sem):
        @pl.when(c < N_CH - 1)
        def _():
            base = pl.multiple_of(c * CH_TOK, CH_TOK)
            for jb in range(4):
                pltpu.async_copy(
                    embT_hbm.at[pl.ds(jb * 8, 8), pl.ds(base, CH_TOK)],
                    inb.at[jb], sem)

        @pl.when(c == N_CH - 1)
        def _():
            base = pl.multiple_of(c * CH_TOK, CH_TOK)
            for jb in range(4):
                pltpu.async_copy(
                    embT_hbm.at[pl.ds(jb * 8, 8), pl.ds(base, 128)],
                    inb.at[jb, :, pl.ds(0, 128)], sem)

    def drain(c, inb, sem):
        @pl.when(c < N_CH - 1)
        def _():
            for jb in range(4):
                pltpu.make_async_copy(
                    embT_hbm.at[pl.ds(0, 8), pl.ds(0, CH_TOK)],
                    inb.at[jb], sem).wait()

        @pl.when(c == N_CH - 1)
        def _():
            for jb in range(4):
                pltpu.make_async_copy(
                    embT_hbm.at[pl.ds(0, 8), pl.ds(0, 128)],
                    inb.at[jb, :, pl.ds(0, 128)], sem).wait()

    RS = D + 1
    iota_rs = iota * RS

    def transpose_chunk(c, inb, row, sem_o):
        def g_body(g, carry):
            gbase = g * L * RS
            for jb in range(4):
                for j in range(8):
                    v = inb[jb, j, pl.ds(g * L, L)]
                    idx = iota_rs + (gbase + jb * 8 + j)
                    plsc.store_scatter(row, [idx], v)
            return carry

        lax.fori_loop(0, CH_TOK // L, g_body, 0)

        def c_body(i, carry):
            v1 = row[pl.ds(i * RS, L)]
            v2 = row[pl.ds(i * RS + L, L)]
            row[pl.ds(i * D, L)] = v1
            row[pl.ds(i * D + L, L)] = v2
            return carry

        lax.fori_loop(0, CH_TOK, c_body, 0, unroll=4)

        @pl.when(c < N_CH - 1)
        def _():
            pltpu.async_copy(
                row.at[pl.ds(0, CH_TOK * D)],
                out_hbm.at[pl.ds(c * (CH_TOK * D), CH_TOK * D)], sem_o)

        @pl.when(c == N_CH - 1)
        def _():
            pltpu.async_copy(
                row.at[pl.ds(0, LAST_TOK * D)],
                out_hbm.at[pl.ds(c * (CH_TOK * D), LAST_TOK * D)], sem_o)

    def drain_out(c, row, sem_o):
        @pl.when(c < N_CH - 1)
        def _():
            pltpu.make_async_copy(
                row.at[pl.ds(0, CH_TOK * D)],
                out_hbm.at[pl.ds(0, CH_TOK * D)], sem_o).wait()

        @pl.when(c == N_CH - 1)
        def _():
            pltpu.make_async_copy(
                row.at[pl.ds(0, LAST_TOK * D)],
                out_hbm.at[pl.ds(0, LAST_TOK * D)], sem_o).wait()

    stage(wid, inb0, sem_i0)
    npairs = (KMAX + 1) // 2

    def k_body(k, carry):
        c0 = wid + NW * (2 * k)
        c1 = wid + NW * (2 * k + 1)
        c2 = wid + NW * (2 * k + 2)
        cp0 = wid + NW * (2 * k - 2)
        cp1 = wid + NW * (2 * k - 1)

        @pl.when(c1 < N_CH)
        def _():
            stage(c1, inb1, sem_i1)

        @pl.when((k > 0) & (cp0 < N_CH))
        def _():
            drain_out(cp0, row0, sem_o0)

        @pl.when(c0 < N_CH)
        def _():
            drain(c0, inb0, sem_i0)
            transpose_chunk(c0, inb0, row0, sem_o0)

        @pl.when(c2 < N_CH)
        def _():
            stage(c2, inb0, sem_i0)

        @pl.when((k > 0) & (cp1 < N_CH))
        def _():
            drain_out(cp1, row1, sem_o1)

        @pl.when(c1 < N_CH)
        def _():
            drain(c1, inb1, sem_i1)
            transpose_chunk(c1, inb1, row1, sem_o1)

        return carry

    lax.fori_loop(0, npairs, k_body, 0)

    @pl.when(wid + NW * (KMAX - 2) < N_CH)
    def _():
        drain_out(wid + NW * (KMAX - 2), row0, sem_o0)

    @pl.when(wid + NW * (KMAX - 1) < N_CH)
    def _():
        drain_out(wid + NW * (KMAX - 1), row1, sem_o1)


def _relayout(embT):
    mesh = plsc.VectorSubcoreMesh(
        core_axis_name="c", subcore_axis_name="s", num_cores=NC, num_subcores=NS)
    run = pl.kernel(
        _relayout_body,
        out_type=jax.ShapeDtypeStruct((VROWS * D,), jnp.float32),
        mesh=mesh,
        compiler_params=pltpu.CompilerParams(
            needs_layout_passes=False, use_tc_tiling_on_sc=True),
        scratch_types=[
            pltpu.VMEM((4, 8, CH_TOK), jnp.float32),
            pltpu.VMEM((4, 8, CH_TOK), jnp.float32),
            pltpu.VMEM((CH_TOK * (D + 1),), jnp.float32),
            pltpu.VMEM((CH_TOK * (D + 1),), jnp.float32),
            pltpu.SemaphoreType.DMA,
            pltpu.SemaphoreType.DMA,
            pltpu.SemaphoreType.DMA,
            pltpu.SemaphoreType.DMA,
        ],
    )
    return run(embT)


def _sc_body(tf_hbm, tg_hbm, emb_hbm, z_hbm, out_hbm,
             tf_v, tg_v, z_v, buf_f0, buf_g0, buf_f1, buf_g1, out_v,
             sem0, sem1):
    cid = lax.axis_index("c")
    sid = lax.axis_index("s")
    wid = sid * NC + cid
    base = wid * RPW

    pltpu.sync_copy(tf_hbm.at[pl.ds(base, RPW), :], tf_v)
    pltpu.sync_copy(tg_hbm.at[pl.ds(base, RPW), :], tg_v)
    pltpu.sync_copy(z_hbm, z_v)
    z0 = z_v[pl.ds(0, L)]
    z1 = z_v[pl.ds(L, L)]

    iota = lax.iota(jnp.int32, L)
    lane0 = iota == 0
    bfly = [jnp.bitwise_xor(iota, k) for k in (8, 4, 2, 1)]

    def issue_pair(r, buf_f, buf_g, sem):
        pltpu.async_copy(
            emb_hbm.at[tf_v.at[r, pl.ds(0, C0)]], buf_f.at[pl.ds(0, C0), :], sem)
        pltpu.async_copy(
            emb_hbm.at[tf_v.at[r, pl.ds(C0, C1)]], buf_f.at[pl.ds(C0, C1), :], sem)
        pltpu.async_copy(
            emb_hbm.at[tg_v.at[r, pl.ds(0, C0)]], buf_g.at[pl.ds(0, C0), :], sem)
        pltpu.async_copy(
            emb_hbm.at[tg_v.at[r, pl.ds(C0, C1)]], buf_g.at[pl.ds(C0, C1), :], sem)

    def drain_pair(buf_f, buf_g, sem):
        pltpu.make_async_copy(emb_hbm.at[pl.ds(0, S), :], buf_f, sem).wait()
        pltpu.make_async_copy(emb_hbm.at[pl.ds(0, S), :], buf_g, sem).wait()

    def compute_row(r, buf_f, buf_g):
        f0a = buf_f[0, pl.ds(0, L)]
        f0b = buf_f[0, pl.ds(L, L)]
        g0a = buf_g[0, pl.ds(0, L)]
        g0b = buf_g[0, pl.ds(L, L)]
        zv = jnp.zeros((L,), jnp.float32)

        def step(s, carry):
            (fsa, fsb, fqa, fqb, fca, fcb, fpa, fpb,
             gsa, gsb, gqa, gqb, gca, gcb, gpa, gpb) = carry
            fva = buf_f[s, pl.ds(0, L)]
            fvb = buf_f[s, pl.ds(L, L)]
            gva = buf_g[s, pl.ds(0, L)]
            gvb = buf_g[s, pl.ds(L, L)]
            return (fsa + fva, fsb + fvb, fqa + fva * fva, fqb + fvb * fvb,
                    fca + fva * fpa, fcb + fvb * fpb, fva, fvb,
                    gsa + gva, gsb + gvb, gqa + gva * gva, gqb + gvb * gvb,
                    gca + gva * gpa, gcb + gvb * gpb, gva, gvb)

        (fsa, fsb, fqa, fqb, fca, fcb, fpa, fpb,
         gsa, gsb, gqa, gqb, gca, gcb, gpa, gpb) = lax.fori_loop(
            0, S, step,
            (zv, zv, zv, zv, zv, zv, f0a, f0b,
             zv, zv, zv, zv, zv, zv, g0a, g0b), unroll=8)

        inv_s = jnp.float32(INV_S)
        inv_d = jnp.float32(INV_D)
        acc = jnp.zeros((L,), jnp.float32)
        for (fs, fq, fc, fp, f0, gs, gq, gc, gp, g0, z) in (
                (fsa, fqa, fca, fpa, f0a, gsa, gqa, gca, gpa, g0a, z0),
                (fsb, fqb, fcb, fpb, f0b, gsb, gqb, gcb, gpb, g0b, z1)):
            sd = (fs - gs) * inv_s
            ed = (fq - gq) * inv_s - 2.0 * z * sd
            dmd = ((fp - f0) - (gp - g0)) * inv_d
            fdq = 2.0 * fq + f0 * f0 - fp * fp - 2.0 * fc
            gdq = 2.0 * gq + g0 * g0 - gp * gp - 2.0 * gc
            ded = (fdq - gdq) * inv_d
            acc = acc + sd * sd + ed * ed + dmd * dmd + ded * ded
        for idx in bfly:
            acc = acc + acc.at[idx].get(mode="promise_in_bounds")
        dist = acc * jnp.float32(1.0 / (4 * D))
        plsc.store_scatter(out_v, [jnp.full((L,), r, jnp.int32)], dist, mask=lane0)

    issue_pair(0, buf_f0, buf_g0, sem0)

    def pair_body(rr, carry):
        r0 = 2 * rr
        issue_pair(r0 + 1, buf_f1, buf_g1, sem1)
        drain_pair(buf_f0, buf_g0, sem0)
        compute_row(r0, buf_f0, buf_g0)

        @pl.when(rr < RPW // 2 - 1)
        def _():
            issue_pair(r0 + 2, buf_f0, buf_g0, sem0)

        drain_pair(buf_f1, buf_g1, sem1)
        compute_row(r0 + 1, buf_f1, buf_g1)
        return carry

    lax.fori_loop(0, RPW // 2, pair_body, 0)
    pltpu.sync_copy(out_v, out_hbm.at[pl.ds(base, RPW)])


def kernel(tokens_f, tokens_g, embedding, state_zero):
    emb_lin = _relayout(embedding.T).reshape(VROWS, D)
    mesh = plsc.VectorSubcoreMesh(
        core_axis_name="c", subcore_axis_name="s", num_cores=NC, num_subcores=NS)
    run = pl.kernel(
        _sc_body,
        out_type=jax.ShapeDtypeStruct((B,), jnp.float32),
        mesh=mesh,
        compiler_params=pltpu.CompilerParams(
            needs_layout_passes=False, use_tc_tiling_on_sc=False),
        scratch_types=[
            pltpu.VMEM((RPW, S), jnp.int32),
            pltpu.VMEM((RPW, S), jnp.int32),
            pltpu.VMEM((D,), jnp.float32),
            pltpu.VMEM((S, D), jnp.float32),
            pltpu.VMEM((S, D), jnp.float32),
            pltpu.VMEM((S, D), jnp.float32),
            pltpu.VMEM((S, D), jnp.float32),
            pltpu.VMEM((RPW,), jnp.float32),
            pltpu.SemaphoreType.DMA,
            pltpu.SemaphoreType.DMA,
        ],
    )
    return run(tokens_f.astype(jnp.int32), tokens_g.astype(jnp.int32),
               emb_lin, state_zero)


# speedup vs baseline: 1.6280x; 1.0086x over previous
import jax
import jax.numpy as jnp
from jax import lax
from jax.experimental import pallas as pl
from jax.experimental.pallas import tpu as pltpu
from jax.experimental.pallas import tpu_sc as plsc

B = 4096
S = 200
D = 32
L = 16
NC = 2
NS = 16
NW = NC * NS
RPW = B // NW
C0 = 128
C1 = S - C0
INV_S = 1.0 / S
INV_D = 1.0 / (S - 1)


VROWS = 1000000
CH_TOK = 512
N_CH = 1954
LAST_TOK = VROWS - (N_CH - 1) * CH_TOK
KMAX = (N_CH + NW - 1) // NW


def _relayout_body(embT_hbm, out_hbm, inb0, inb1, row0, row1, cmp0, cmp1,
                   sem_i0, sem_i1, sem_o0, sem_o1):
    cid = lax.axis_index("c")
    sid = lax.axis_index("s")
    wid = sid * NC + cid

    iota = lax.iota(jnp.int32, L)
    jb_lo = lax.shift_right_logical(iota, 3)
    jb_hi = jb_lo + 2
    j_lane = jnp.bitwise_and(iota, 7)

    def stage(c, inb, sem):
        @pl.when(c < N_CH - 1)
        def _():
            base = pl.multiple_of(c * CH_TOK, CH_TOK)
            for jb in range(4):
                pltpu.async_copy(
                    embT_hbm.at[pl.ds(jb * 8, 8), pl.ds(base, CH_TOK)],
                    inb.at[jb], sem)

        @pl.when(c == N_CH - 1)
        def _():
            base = pl.multiple_of(c * CH_TOK, CH_TOK)
            for jb in range(4):
                pltpu.async_copy(
                    embT_hbm.at[pl.ds(jb * 8, 8), pl.ds(base, 128)],
                    inb.at[jb, :, pl.ds(0, 128)], sem)

    def drain(c, inb, sem):
        @pl.when(c < N_CH - 1)
        def _():
            for jb in range(4):
                pltpu.make_async_copy(
                    embT_hbm.at[pl.ds(0, 8), pl.ds(0, CH_TOK)],
                    inb.at[jb], sem).wait()

        @pl.when(c == N_CH - 1)
        def _():
            for jb in range(4):
                pltpu.make_async_copy(
                    embT_hbm.at[pl.ds(0, 8), pl.ds(0, 128)],
                    inb.at[jb, :, pl.ds(0, 128)], sem).wait()

    RS = D + 1
    iota_rs = iota * RS

    def transpose_chunk(c, inb, row, cmp, sem_o):
        def g_body(g, carry):
            gbase = g * L * RS
            for jb in range(4):
                for j in range(8):
                    v = inb[jb, j, pl.ds(g * L, L)]
                    idx = iota_rs + (gbase + jb * 8 + j)
                    plsc.store_scatter(row, [idx], v)
            return carry

        lax.fori_loop(0, CH_TOK // L, g_body, 0)

        def c_body(i, carry):
            v1 = row[pl.ds(i * RS, L)]
            v2 = row[pl.ds(i * RS + L, L)]
            cmp[pl.ds(i * D, L)] = v1
            cmp[pl.ds(i * D + L, L)] = v2
            return carry

        lax.fori_loop(0, CH_TOK, c_body, 0, unroll=8)

        @pl.when(c < N_CH - 1)
        def _():
            pltpu.async_copy(
                cmp, out_hbm.at[pl.ds(c * (CH_TOK * D), CH_TOK * D)], sem_o)

        @pl.when(c == N_CH - 1)
        def _():
            pltpu.async_copy(
                cmp.at[pl.ds(0, LAST_TOK * D)],
                out_hbm.at[pl.ds(c * (CH_TOK * D), LAST_TOK * D)], sem_o)

    def drain_out(c, cmp, sem_o):
        @pl.when(c < N_CH - 1)
        def _():
            pltpu.make_async_copy(
                cmp, out_hbm.at[pl.ds(0, CH_TOK * D)], sem_o).wait()

        @pl.when(c == N_CH - 1)
        def _():
            pltpu.make_async_copy(
                cmp.at[pl.ds(0, LAST_TOK * D)],
                out_hbm.at[pl.ds(0, LAST_TOK * D)], sem_o).wait()

    stage(wid, inb0, sem_i0)
    npairs = (KMAX + 1) // 2

    def k_body(k, carry):
        c0 = wid + NW * (2 * k)
        c1 = wid + NW * (2 * k + 1)
        c2 = wid + NW * (2 * k + 2)
        cp0 = wid + NW * (2 * k - 2)
        cp1 = wid + NW * (2 * k - 1)

        @pl.when(c1 < N_CH)
        def _():
            stage(c1, inb1, sem_i1)

        @pl.when((k > 0) & (cp0 < N_CH))
        def _():
            drain_out(cp0, cmp0, sem_o0)

        @pl.when(c0 < N_CH)
        def _():
            drain(c0, inb0, sem_i0)
            transpose_chunk(c0, inb0, row0, cmp0, sem_o0)

        @pl.when(c2 < N_CH)
        def _():
            stage(c2, inb0, sem_i0)

        @pl.when((k > 0) & (cp1 < N_CH))
        def _():
            drain_out(cp1, cmp1, sem_o1)

        @pl.when(c1 < N_CH)
        def _():
            drain(c1, inb1, sem_i1)
            transpose_chunk(c1, inb1, row1, cmp1, sem_o1)

        return carry

    lax.fori_loop(0, npairs, k_body, 0)

    @pl.when(wid + NW * (KMAX - 2) < N_CH)
    def _():
        drain_out(wid + NW * (KMAX - 2), cmp0, sem_o0)

    @pl.when(wid + NW * (KMAX - 1) < N_CH)
    def _():
        drain_out(wid + NW * (KMAX - 1), cmp1, sem_o1)


def _relayout(embT):
    mesh = plsc.VectorSubcoreMesh(
        core_axis_name="c", subcore_axis_name="s", num_cores=NC, num_subcores=NS)
    run = pl.kernel(
        _relayout_body,
        out_type=jax.ShapeDtypeStruct((VROWS * D,), jnp.float32),
        mesh=mesh,
        compiler_params=pltpu.CompilerParams(
            needs_layout_passes=False, use_tc_tiling_on_sc=True),
        scratch_types=[
            pltpu.VMEM((4, 8, CH_TOK), jnp.float32),
            pltpu.VMEM((4, 8, CH_TOK), jnp.float32),
            pltpu.VMEM((CH_TOK * (D + 1),), jnp.float32),
            pltpu.VMEM((CH_TOK * (D + 1),), jnp.float32),
            pltpu.VMEM((CH_TOK * D,), jnp.float32),
            pltpu.VMEM((CH_TOK * D,), jnp.float32),
            pltpu.SemaphoreType.DMA,
            pltpu.SemaphoreType.DMA,
            pltpu.SemaphoreType.DMA,
            pltpu.SemaphoreType.DMA,
        ],
    )
    return run(embT)


def _sc_body(tf_hbm, tg_hbm, emb_hbm, z_hbm, out_hbm,
             tf_v, tg_v, z_v, buf_f0, buf_g0, buf_f1, buf_g1, out_v,
             sem0, sem1):
    cid = lax.axis_index("c")
    sid = lax.axis_index("s")
    wid = sid * NC + cid
    base = wid * RPW

    pltpu.sync_copy(tf_hbm.at[pl.ds(base, RPW), :], tf_v)
    pltpu.sync_copy(tg_hbm.at[pl.ds(base, RPW), :], tg_v)
    pltpu.sync_copy(z_hbm, z_v)
    z0 = z_v[pl.ds(0, L)]
    z1 = z_v[pl.ds(L, L)]

    iota = lax.iota(jnp.int32, L)
    lane0 = iota == 0
    bfly = [jnp.bitwise_xor(iota, k) for k in (8, 4, 2, 1)]

    def issue_pair(r, buf_f, buf_g, sem):
        pltpu.async_copy(
            emb_hbm.at[tf_v.at[r, pl.ds(0, C0)]], buf_f.at[pl.ds(0, C0), :], sem)
        pltpu.async_copy(
            emb_hbm.at[tf_v.at[r, pl.ds(C0, C1)]], buf_f.at[pl.ds(C0, C1), :], sem)
        pltpu.async_copy(
            emb_hbm.at[tg_v.at[r, pl.ds(0, C0)]], buf_g.at[pl.ds(0, C0), :], sem)
        pltpu.async_copy(
            emb_hbm.at[tg_v.at[r, pl.ds(C0, C1)]], buf_g.at[pl.ds(C0, C1), :], sem)

    def drain_pair(buf_f, buf_g, sem):
        pltpu.make_async_copy(emb_hbm.at[pl.ds(0, S), :], buf_f, sem).wait()
        pltpu.make_async_copy(emb_hbm.at[pl.ds(0, S), :], buf_g, sem).wait()

    def compute_row(r, buf_f, buf_g):
        f0a = buf_f[0, pl.ds(0, L)]
        f0b = buf_f[0, pl.ds(L, L)]
        g0a = buf_g[0, pl.ds(0, L)]
        g0b = buf_g[0, pl.ds(L, L)]
        zv = jnp.zeros((L,), jnp.float32)

        def step(s, carry):
            (fsa, fsb, fqa, fqb, fca, fcb, fpa, fpb,
             gsa, gsb, gqa, gqb, gca, gcb, gpa, gpb) = carry
            fva = buf_f[s, pl.ds(0, L)]
            fvb = buf_f[s, pl.ds(L, L)]
            gva = buf_g[s, pl.ds(0, L)]
            gvb = buf_g[s, pl.ds(L, L)]
            return (fsa + fva, fsb + fvb, fqa + fva * fva, fqb + fvb * fvb,
                    fca + fva * fpa, fcb + fvb * fpb, fva, fvb,
                    gsa + gva, gsb + gvb, gqa + gva * gva, gqb + gvb * gvb,
                    gca + gva * gpa, gcb + gvb * gpb, gva, gvb)

        (fsa, fsb, fqa, fqb, fca, fcb, fpa, fpb,
         gsa, gsb, gqa, gqb, gca, gcb, gpa, gpb) = lax.fori_loop(
            0, S, step,
            (zv, zv, zv, zv, zv, zv, f0a, f0b,
             zv, zv, zv, zv, zv, zv, g0a, g0b), unroll=8)

        inv_s = jnp.float32(INV_S)
        inv_d = jnp.float32(INV_D)
        acc = jnp.zeros((L,), jnp.float32)
        for (fs, fq, fc, fp, f0, gs, gq, gc, gp, g0, z) in (
                (fsa, fqa, fca, fpa, f0a, gsa, gqa, gca, gpa, g0a, z0),
                (fsb, fqb, fcb, fpb, f0b, gsb, gqb, gcb, gpb, g0b, z1)):
            sd = (fs - gs) * inv_s
            ed = (fq - gq) * inv_s - 2.0 * z * sd
            dmd = ((fp - f0) - (gp - g0)) * inv_d
            fdq = 2.0 * fq + f0 * f0 - fp * fp - 2.0 * fc
            gdq = 2.0 * gq + g0 * g0 - gp * gp - 2.0 * gc
            ded = (fdq - gdq) * inv_d
            acc = acc + sd * sd + ed * ed + dmd * dmd + ded * ded
        for idx in bfly:
            acc = acc + acc.at[idx].get(mode="promise_in_bounds")
        dist = acc * jnp.float32(1.0 / (4 * D))
        plsc.store_scatter(out_v, [jnp.full((L,), r, jnp.int32)], dist, mask=lane0)

    issue_pair(0, buf_f0, buf_g0, sem0)

    def pair_body(rr, carry):
        r0 = 2 * rr
        issue_pair(r0 + 1, buf_f1, buf_g1, sem1)
        drain_pair(buf_f0, buf_g0, sem0)
        compute_row(r0, buf_f0, buf_g0)

        @pl.when(rr < RPW // 2 - 1)
        def _():
            issue_pair(r0 + 2, buf_f0, buf_g0, sem0)

        drain_pair(buf_f1, buf_g1, sem1)
        compute_row(r0 + 1, buf_f1, buf_g1)
        return carry

    lax.fori_loop(0, RPW // 2, pair_body, 0)
    pltpu.sync_copy(out_v, out_hbm.at[pl.ds(base, RPW)])


def kernel(tokens_f, tokens_g, embedding, state_zero):
    emb_lin = _relayout(embedding.T).reshape(VROWS, D)
    mesh = plsc.VectorSubcoreMesh(
        core_axis_name="c", subcore_axis_name="s", num_cores=NC, num_subcores=NS)
    run = pl.kernel(
        _sc_body,
        out_type=jax.ShapeDtypeStruct((B,), jnp.float32),
        mesh=mesh,
        compiler_params=pltpu.CompilerParams(
            needs_layout_passes=False, use_tc_tiling_on_sc=False),
        scratch_types=[
            pltpu.VMEM((RPW, S), jnp.int32),
            pltpu.VMEM((RPW, S), jnp.int32),
            pltpu.VMEM((D,), jnp.float32),
            pltpu.VMEM((S, D), jnp.float32),
            pltpu.VMEM((S, D), jnp.float32),
            pltpu.VMEM((S, D), jnp.float32),
            pltpu.VMEM((S, D), jnp.float32),
            pltpu.VMEM((RPW,), jnp.float32),
            pltpu.SemaphoreType.DMA,
            pltpu.SemaphoreType.DMA,
        ],
    )
    return run(tokens_f.astype(jnp.int32), tokens_g.astype(jnp.int32),
               emb_lin, state_zero)


# speedup vs baseline: 1.6290x; 1.0006x over previous
import jax
import jax.numpy as jnp
from jax import lax
from jax.experimental import pallas as pl
from jax.experimental.pallas import tpu as pltpu
from jax.experimental.pallas import tpu_sc as plsc

B = 4096
S = 200
D = 32
L = 16
NC = 2
NS = 16
NW = NC * NS
RPW = B // NW
RS = D + 1
C0 = 128
C1 = S - C0
INV_S = 1.0 / S
INV_D = 1.0 / (S - 1)


VROWS = 1000000
CH_TOK = 512
N_CH = 1954
LAST_TOK = VROWS - (N_CH - 1) * CH_TOK
_KM = (N_CH + NW - 1) // NW
KMAX = _KM + (_KM % 2)


def _relayout_body(embT_hbm, out_hbm, inb0, inb1, row0, row1, cmp0, cmp1,
                   sem_i0, sem_i1, sem_o0, sem_o1):
    cid = lax.axis_index("c")
    sid = lax.axis_index("s")
    wid = sid * NC + cid

    iota = lax.iota(jnp.int32, L)
    jb_lo = lax.shift_right_logical(iota, 3)
    jb_hi = jb_lo + 2
    j_lane = jnp.bitwise_and(iota, 7)

    def stage(c, inb, sem):
        @pl.when(c < N_CH - 1)
        def _():
            base = pl.multiple_of(c * CH_TOK, CH_TOK)
            for jb in range(4):
                pltpu.async_copy(
                    embT_hbm.at[pl.ds(jb * 8, 8), pl.ds(base, CH_TOK)],
                    inb.at[jb], sem)

        @pl.when(c == N_CH - 1)
        def _():
            base = pl.multiple_of(c * CH_TOK, CH_TOK)
            for jb in range(4):
                pltpu.async_copy(
                    embT_hbm.at[pl.ds(jb * 8, 8), pl.ds(base, 128)],
                    inb.at[jb, :, pl.ds(0, 128)], sem)

    def drain(c, inb, sem):
        @pl.when(c < N_CH - 1)
        def _():
            for jb in range(4):
                pltpu.make_async_copy(
                    embT_hbm.at[pl.ds(0, 8), pl.ds(0, CH_TOK)],
                    inb.at[jb], sem).wait()

        @pl.when(c == N_CH - 1)
        def _():
            for jb in range(4):
                pltpu.make_async_copy(
                    embT_hbm.at[pl.ds(0, 8), pl.ds(0, 128)],
                    inb.at[jb, :, pl.ds(0, 128)], sem).wait()

    iota_rs = iota * RS

    def transpose_chunk(c, inb, row, cmp, sem_o):
        def g_body(g, carry):
            gbase = g * L * RS
            for jb in range(4):
                for j in range(8):
                    v = inb[jb, j, pl.ds(g * L, L)]
                    idx = iota_rs + (gbase + jb * 8 + j)
                    plsc.store_scatter(row, [idx], v)
            return carry

        lax.fori_loop(0, CH_TOK // L, g_body, 0)

        def c_body(i, carry):
            v1 = row[pl.ds(i * RS, L)]
            v2 = row[pl.ds(i * RS + L, L)]
            cmp[pl.ds(i * D, L)] = v1
            cmp[pl.ds(i * D + L, L)] = v2
            return carry

        lax.fori_loop(0, CH_TOK, c_body, 0, unroll=8)

        @pl.when(c < N_CH - 1)
        def _():
            pltpu.async_copy(
                cmp, out_hbm.at[pl.ds(c * (CH_TOK * D), CH_TOK * D)], sem_o)

        @pl.when(c == N_CH - 1)
        def _():
            pltpu.async_copy(
                cmp.at[pl.ds(0, LAST_TOK * D)],
                out_hbm.at[pl.ds(c * (CH_TOK * D), LAST_TOK * D)], sem_o)

    def drain_out(c, cmp, sem_o):
        @pl.when(c < N_CH - 1)
        def _():
            pltpu.make_async_copy(
                cmp, out_hbm.at[pl.ds(0, CH_TOK * D)], sem_o).wait()

        @pl.when(c == N_CH - 1)
        def _():
            pltpu.make_async_copy(
                cmp.at[pl.ds(0, LAST_TOK * D)],
                out_hbm.at[pl.ds(0, LAST_TOK * D)], sem_o).wait()

    stage(wid, inb0, sem_i0)
    npairs = (KMAX + 1) // 2

    def k_body(k, carry):
        c0 = wid + NW * (2 * k)
        c1 = wid + NW * (2 * k + 1)
        c2 = wid + NW * (2 * k + 2)
        cp0 = wid + NW * (2 * k - 2)
        cp1 = wid + NW * (2 * k - 1)

        @pl.when(c1 < N_CH)
        def _():
            stage(c1, inb1, sem_i1)

        @pl.when((k > 0) & (cp0 < N_CH))
        def _():
            drain_out(cp0, cmp0, sem_o0)

        @pl.when(c0 < N_CH)
        def _():
            drain(c0, inb0, sem_i0)
            transpose_chunk(c0, inb0, row0, cmp0, sem_o0)

        @pl.when(c2 < N_CH)
        def _():
            stage(c2, inb0, sem_i0)

        @pl.when((k > 0) & (cp1 < N_CH))
        def _():
            drain_out(cp1, cmp1, sem_o1)

        @pl.when(c1 < N_CH)
        def _():
            drain(c1, inb1, sem_i1)
            transpose_chunk(c1, inb1, row1, cmp1, sem_o1)

        return carry

    lax.fori_loop(0, npairs, k_body, 0)

    @pl.when(wid + NW * (KMAX - 2) < N_CH)
    def _():
        drain_out(wid + NW * (KMAX - 2), cmp0, sem_o0)

    @pl.when(wid + NW * (KMAX - 1) < N_CH)
    def _():
        drain_out(wid + NW * (KMAX - 1), cmp1, sem_o1)


def _relayout(embT):
    mesh = plsc.VectorSubcoreMesh(
        core_axis_name="c", subcore_axis_name="s", num_cores=NC, num_subcores=NS)
    run = pl.kernel(
        _relayout_body,
        out_type=jax.ShapeDtypeStruct((VROWS * D,), jnp.float32),
        mesh=mesh,
        compiler_params=pltpu.CompilerParams(
            needs_layout_passes=False, use_tc_tiling_on_sc=True),
        scratch_types=[
            pltpu.VMEM((4, 8, CH_TOK), jnp.float32),
            pltpu.VMEM((4, 8, CH_TOK), jnp.float32),
            pltpu.VMEM((CH_TOK * RS,), jnp.float32),
            pltpu.VMEM((CH_TOK * RS,), jnp.float32),
            pltpu.VMEM((CH_TOK * D,), jnp.float32),
            pltpu.VMEM((CH_TOK * D,), jnp.float32),
            pltpu.SemaphoreType.DMA,
            pltpu.SemaphoreType.DMA,
            pltpu.SemaphoreType.DMA,
            pltpu.SemaphoreType.DMA,
        ],
    )
    return run(embT)


def _sc_body(tf_hbm, tg_hbm, emb_hbm, z_hbm, out_hbm,
             tf_v, tg_v, z_v, buf_f0, buf_g0, buf_f1, buf_g1, out_v,
             sem0, sem1):
    cid = lax.axis_index("c")
    sid = lax.axis_index("s")
    wid = sid * NC + cid
    base = wid * RPW

    pltpu.sync_copy(tf_hbm.at[pl.ds(base, RPW), :], tf_v)
    pltpu.sync_copy(tg_hbm.at[pl.ds(base, RPW), :], tg_v)
    pltpu.sync_copy(z_hbm, z_v)
    z0 = z_v[pl.ds(0, L)]
    z1 = z_v[pl.ds(L, L)]

    iota = lax.iota(jnp.int32, L)
    lane0 = iota == 0
    bfly = [jnp.bitwise_xor(iota, k) for k in (8, 4, 2, 1)]

    def issue_pair(r, buf_f, buf_g, sem):
        pltpu.async_copy(
            emb_hbm.at[tf_v.at[r, pl.ds(0, C0)]], buf_f.at[pl.ds(0, C0), :], sem)
        pltpu.async_copy(
            emb_hbm.at[tf_v.at[r, pl.ds(C0, C1)]], buf_f.at[pl.ds(C0, C1), :], sem)
        pltpu.async_copy(
            emb_hbm.at[tg_v.at[r, pl.ds(0, C0)]], buf_g.at[pl.ds(0, C0), :], sem)
        pltpu.async_copy(
            emb_hbm.at[tg_v.at[r, pl.ds(C0, C1)]], buf_g.at[pl.ds(C0, C1), :], sem)

    def drain_pair(buf_f, buf_g, sem):
        pltpu.make_async_copy(emb_hbm.at[pl.ds(0, S), :], buf_f, sem).wait()
        pltpu.make_async_copy(emb_hbm.at[pl.ds(0, S), :], buf_g, sem).wait()

    def compute_row(r, buf_f, buf_g):
        f0a = buf_f[0, pl.ds(0, L)]
        f0b = buf_f[0, pl.ds(L, L)]
        g0a = buf_g[0, pl.ds(0, L)]
        g0b = buf_g[0, pl.ds(L, L)]
        zv = jnp.zeros((L,), jnp.float32)

        def step(s, carry):
            (fsa, fsb, fqa, fqb, fca, fcb, fpa, fpb,
             gsa, gsb, gqa, gqb, gca, gcb, gpa, gpb) = carry
            fva = buf_f[s, pl.ds(0, L)]
            fvb = buf_f[s, pl.ds(L, L)]
            gva = buf_g[s, pl.ds(0, L)]
            gvb = buf_g[s, pl.ds(L, L)]
            return (fsa + fva, fsb + fvb, fqa + fva * fva, fqb + fvb * fvb,
                    fca + fva * fpa, fcb + fvb * fpb, fva, fvb,
                    gsa + gva, gsb + gvb, gqa + gva * gva, gqb + gvb * gvb,
                    gca + gva * gpa, gcb + gvb * gpb, gva, gvb)

        (fsa, fsb, fqa, fqb, fca, fcb, fpa, fpb,
         gsa, gsb, gqa, gqb, gca, gcb, gpa, gpb) = lax.fori_loop(
            0, S, step,
            (zv, zv, zv, zv, zv, zv, f0a, f0b,
             zv, zv, zv, zv, zv, zv, g0a, g0b), unroll=8)

        inv_s = jnp.float32(INV_S)
        inv_d = jnp.float32(INV_D)
        acc = jnp.zeros((L,), jnp.float32)
        for (fs, fq, fc, fp, f0, gs, gq, gc, gp, g0, z) in (
                (fsa, fqa, fca, fpa, f0a, gsa, gqa, gca, gpa, g0a, z0),
                (fsb, fqb, fcb, fpb, f0b, gsb, gqb, gcb, gpb, g0b, z1)):
            sd = (fs - gs) * inv_s
            ed = (fq - gq) * inv_s - 2.0 * z * sd
            dmd = ((fp - f0) - (gp - g0)) * inv_d
            fdq = 2.0 * fq + f0 * f0 - fp * fp - 2.0 * fc
            gdq = 2.0 * gq + g0 * g0 - gp * gp - 2.0 * gc
            ded = (fdq - gdq) * inv_d
            acc = acc + sd * sd + ed * ed + dmd * dmd + ded * ded
        for idx in bfly:
            acc = acc + acc.at[idx].get(mode="promise_in_bounds")
        dist = acc * jnp.float32(1.0 / (4 * D))
        plsc.store_scatter(out_v, [jnp.full((L,), r, jnp.int32)], dist, mask=lane0)

    issue_pair(0, buf_f0, buf_g0, sem0)

    def pair_body(rr, carry):
        r0 = 2 * rr
        issue_pair(r0 + 1, buf_f1, buf_g1, sem1)
        drain_pair(buf_f0, buf_g0, sem0)
        compute_row(r0, buf_f0, buf_g0)

        @pl.when(rr < RPW // 2 - 1)
        def _():
            issue_pair(r0 + 2, buf_f0, buf_g0, sem0)

        drain_pair(buf_f1, buf_g1, sem1)
        compute_row(r0 + 1, buf_f1, buf_g1)
        return carry

    lax.fori_loop(0, RPW // 2, pair_body, 0)
    pltpu.sync_copy(out_v, out_hbm.at[pl.ds(base, RPW)])


def kernel(tokens_f, tokens_g, embedding, state_zero):
    emb_lin = _relayout(embedding.T).reshape(VROWS, D)
    mesh = plsc.VectorSubcoreMesh(
        core_axis_name="c", subcore_axis_name="s", num_cores=NC, num_subcores=NS)
    run = pl.kernel(
        _sc_body,
        out_type=jax.ShapeDtypeStruct((B,), jnp.float32),
        mesh=mesh,
        compiler_params=pltpu.CompilerParams(
            needs_layout_passes=False, use_tc_tiling_on_sc=False),
        scratch_types=[
            pltpu.VMEM((RPW, S), jnp.int32),
            pltpu.VMEM((RPW, S), jnp.int32),
            pltpu.VMEM((D,), jnp.float32),
            pltpu.VMEM((S, D), jnp.float32),
            pltpu.VMEM((S, D), jnp.float32),
            pltpu.VMEM((S, D), jnp.float32),
            pltpu.VMEM((S, D), jnp.float32),
            pltpu.VMEM((RPW,), jnp.float32),
            pltpu.SemaphoreType.DMA,
            pltpu.SemaphoreType.DMA,
        ],
    )
    return run(tokens_f.astype(jnp.int32), tokens_g.astype(jnp.int32),
               emb_lin, state_zero)
